# Initial kernel scaffold; baseline (speedup 1.0000x reference)
import jax
import jax.numpy as jnp
from jax.experimental import pallas as pl


def kernel(x, Wg, bg, W1, b1, g1, be1, W2, b2):
    raise NotImplementedError("write your pallas kernel here")


# speedup vs baseline: 9.2505x; 9.2505x over previous
import functools

import jax
import jax.numpy as jnp
from jax.experimental import pallas as pl
from jax.experimental.pallas import tpu as pltpu

_N_TOKENS = 4096
_D_MODEL = 1024
_N_EXPERTS = 8
_HIDDEN = 128
_OUT_DIM = 1024
_TOP_K = 2
_TILE = 512
_ACC_W = 128


def _moe_body(x_ref, Wg_ref, bg_ref, W1_ref, b1_ref, g1_ref, be1_ref,
              W2_ref, b2_ref, out_ref, aux_ref, acc_ref):
    i = pl.program_id(0)
    nsteps = pl.num_programs(0)
    E = _N_EXPERTS

    x = x_ref[...]

    gs = jnp.dot(x, Wg_ref[...], preferred_element_type=jnp.float32) + bg_ref[...]
    iota = jax.lax.broadcasted_iota(jnp.int32, gs.shape, 1)
    v1 = jnp.max(gs, axis=1, keepdims=True)
    idx1 = jnp.min(jnp.where(gs >= v1, iota, E), axis=1, keepdims=True)
    sel1 = iota == idx1
    gs_m = jnp.where(sel1, -jnp.inf, gs)
    v2 = jnp.max(gs_m, axis=1, keepdims=True)
    idx2 = jnp.min(jnp.where(gs_m >= v2, iota, E), axis=1, keepdims=True)
    sel2 = iota == idx2
    e2 = jnp.exp(v2 - v1)
    denom = 1.0 + e2
    w = jnp.where(sel1, 1.0 / denom, 0.0) + jnp.where(sel2, e2 / denom, 0.0)

    ex = jnp.exp(gs - v1)
    se = jnp.sum(ex, axis=1, keepdims=True)
    lse = jnp.log(se) + v1
    logp = gs - lse
    p = jnp.exp(logp)
    ent = -jnp.sum(p * logp, axis=1, keepdims=True)
    counts = jnp.sum(jnp.where(sel1 | sel2, 1.0, 0.0), axis=0, keepdims=True)
    ent_sum = jnp.sum(ent, axis=0, keepdims=True)
    part = jnp.concatenate(
        [counts, ent_sum, jnp.zeros((1, _ACC_W - E - 1), jnp.float32)], axis=1)

    @pl.when(i == 0)
    def _():
        acc_ref[...] = jnp.zeros_like(acc_ref)

    acc_ref[...] += part

    @pl.when(i == nsteps - 1)
    def _():
        acc = acc_ref[...]
        usage = acc[:, 0:E] / _N_TOKENS
        lb = jnp.mean((usage - 1.0 / E) ** 2)
        ent_mean = acc[0, E] / _N_TOKENS
        aux_ref[...] = jnp.full((1, 1), lb - 0.1 * ent_mean, jnp.float32)

    acc_out = jnp.dot(w, b2_ref[...], preferred_element_type=jnp.float32)
    for e in range(E):
        h = jnp.dot(x, W1_ref[e], preferred_element_type=jnp.float32) + b1_ref[e]
        h = 0.5 * h * (1.0 + jax.lax.erf(h * 0.7071067811865476))
        mu = jnp.mean(h, axis=1, keepdims=True)
        d = h - mu
        var = jnp.mean(d * d, axis=1, keepdims=True)
        hn = d / jnp.sqrt(var + 1e-5) * g1_ref[e] + be1_ref[e]
        hw = hn * w[:, e:e + 1]
        acc_out += jnp.dot(hw, W2_ref[e], preferred_element_type=jnp.float32)
    out_ref[...] = acc_out


@jax.jit
def kernel(x, Wg, bg, W1, b1, g1, be1, W2, b2):
    T = _TILE
    grid = _N_TOKENS // T
    out, aux = pl.pallas_call(
        _moe_body,
        grid=(grid,),
        in_specs=[
            pl.BlockSpec((T, _D_MODEL), lambda i: (i, 0)),
            pl.BlockSpec((_D_MODEL, _N_EXPERTS), lambda i: (0, 0)),
            pl.BlockSpec((1, _N_EXPERTS), lambda i: (0, 0)),
            pl.BlockSpec((_N_EXPERTS, _D_MODEL, _HIDDEN), lambda i: (0, 0, 0)),
            pl.BlockSpec((_N_EXPERTS, _HIDDEN), lambda i: (0, 0)),
            pl.BlockSpec((_N_EXPERTS, _HIDDEN), lambda i: (0, 0)),
            pl.BlockSpec((_N_EXPERTS, _HIDDEN), lambda i: (0, 0)),
            pl.BlockSpec((_N_EXPERTS, _HIDDEN, _OUT_DIM), lambda i: (0, 0, 0)),
            pl.BlockSpec((_N_EXPERTS, _OUT_DIM), lambda i: (0, 0)),
        ],
        out_specs=[
            pl.BlockSpec((T, _OUT_DIM), lambda i: (i, 0)),
            pl.BlockSpec((1, 1), lambda i: (0, 0)),
        ],
        out_shape=[
            jax.ShapeDtypeStruct((_N_TOKENS, _OUT_DIM), jnp.float32),
            jax.ShapeDtypeStruct((1, 1), jnp.float32),
        ],
        scratch_shapes=[pltpu.VMEM((1, _ACC_W), jnp.float32)],
        compiler_params=pltpu.CompilerParams(
            dimension_semantics=("arbitrary",)),
    )(x, Wg, bg.reshape(1, -1), W1, b1, g1, be1, W2, b2)
    return out, aux[0, 0]


# speedup vs baseline: 15.6738x; 1.6944x over previous
import jax
import jax.numpy as jnp
from jax.experimental import pallas as pl
from jax.experimental.pallas import tpu as pltpu

_N_TOKENS = 4096
_D_MODEL = 1024
_N_EXPERTS = 8
_HIDDEN = 128
_OUT_DIM = 1024
_TILE = 512
_ACC_W = 128


def _moe_body(x_ref, Wg_ref, bg_ref, W1p_ref, b1_ref, g1_ref, be1_ref,
              W2p_ref, b2_ref, out_ref, aux_ref, acc_ref):
    i = pl.program_id(0)
    nsteps = pl.num_programs(0)
    E = _N_EXPERTS
    H = _HIDDEN

    x = x_ref[...]

    gs = jnp.dot(x, Wg_ref[...], preferred_element_type=jnp.float32) + bg_ref[...]
    iota = jax.lax.broadcasted_iota(jnp.int32, gs.shape, 1)
    v1 = jnp.max(gs, axis=1, keepdims=True)
    idx1 = jnp.min(jnp.where(gs >= v1, iota, E), axis=1, keepdims=True)
    sel1 = iota == idx1
    gs_m = jnp.where(sel1, -jnp.inf, gs)
    v2 = jnp.max(gs_m, axis=1, keepdims=True)
    idx2 = jnp.min(jnp.where(gs_m >= v2, iota, E), axis=1, keepdims=True)
    sel2 = iota == idx2
    e2 = jnp.exp(v2 - v1)
    denom = 1.0 + e2
    w = jnp.where(sel1, 1.0 / denom, 0.0) + jnp.where(sel2, e2 / denom, 0.0)

    ex = jnp.exp(gs - v1)
    se = jnp.sum(ex, axis=1, keepdims=True)
    lse = jnp.log(se) + v1
    logp = gs - lse
    p = jnp.exp(logp)
    ent = -jnp.sum(p * logp, axis=1, keepdims=True)
    counts = jnp.sum(jnp.where(sel1 | sel2, 1.0, 0.0), axis=0, keepdims=True)
    ent_sum = jnp.sum(ent, axis=0, keepdims=True)
    part = jnp.concatenate(
        [counts, ent_sum, jnp.zeros((1, _ACC_W - E - 1), jnp.float32)], axis=1)

    @pl.when(i == 0)
    def _():
        acc_ref[...] = jnp.zeros_like(acc_ref)

    acc_ref[...] += part

    @pl.when(i == nsteps - 1)
    def _():
        acc = acc_ref[...]
        usage = acc[:, 0:E] / _N_TOKENS
        lb = jnp.mean((usage - 1.0 / E) ** 2)
        ent_mean = acc[0, E] / _N_TOKENS
        aux_ref[...] = jnp.full((1, 1), lb - 0.1 * ent_mean, jnp.float32)

    xb = x.astype(jnp.bfloat16)
    h_all = jnp.dot(xb, W1p_ref[...], preferred_element_type=jnp.float32)
    h_all += b1_ref[...]
    h_all = 0.5 * h_all * (1.0 + jax.lax.erf(h_all * 0.7071067811865476))
    g1 = g1_ref[...]
    be1 = be1_ref[...]
    parts = []
    for e in range(E):
        he = h_all[:, e * H:(e + 1) * H]
        mu = jnp.mean(he, axis=1, keepdims=True)
        d = he - mu
        var = jnp.mean(d * d, axis=1, keepdims=True)
        hn = d / jnp.sqrt(var + 1e-5) * g1[:, e * H:(e + 1) * H] + be1[:, e * H:(e + 1) * H]
        parts.append((hn * w[:, e:e + 1]).astype(jnp.bfloat16))
    hw_all = jnp.concatenate(parts, axis=1)
    acc_out = jnp.dot(hw_all, W2p_ref[...], preferred_element_type=jnp.float32)
    acc_out += jnp.dot(w, b2_ref[...], preferred_element_type=jnp.float32)
    out_ref[...] = acc_out


@jax.jit
def kernel(x, Wg, bg, W1, b1, g1, be1, W2, b2):
    T = _TILE
    grid = _N_TOKENS // T
    EH = _N_EXPERTS * _HIDDEN
    W1p = jnp.transpose(W1, (1, 0, 2)).reshape(_D_MODEL, EH).astype(jnp.bfloat16)
    W2p = W2.reshape(EH, _OUT_DIM).astype(jnp.bfloat16)
    out, aux = pl.pallas_call(
        _moe_body,
        grid=(grid,),
        in_specs=[
            pl.BlockSpec((T, _D_MODEL), lambda i: (i, 0)),
            pl.BlockSpec((_D_MODEL, _N_EXPERTS), lambda i: (0, 0)),
            pl.BlockSpec((1, _N_EXPERTS), lambda i: (0, 0)),
            pl.BlockSpec((_D_MODEL, EH), lambda i: (0, 0)),
            pl.BlockSpec((1, EH), lambda i: (0, 0)),
            pl.BlockSpec((1, EH), lambda i: (0, 0)),
            pl.BlockSpec((1, EH), lambda i: (0, 0)),
            pl.BlockSpec((EH, _OUT_DIM), lambda i: (0, 0)),
            pl.BlockSpec((_N_EXPERTS, _OUT_DIM), lambda i: (0, 0)),
        ],
        out_specs=[
            pl.BlockSpec((T, _OUT_DIM), lambda i: (i, 0)),
            pl.BlockSpec((1, 1), lambda i: (0, 0)),
        ],
        out_shape=[
            jax.ShapeDtypeStruct((_N_TOKENS, _OUT_DIM), jnp.float32),
            jax.ShapeDtypeStruct((1, 1), jnp.float32),
        ],
        scratch_shapes=[pltpu.VMEM((1, _ACC_W), jnp.float32)],
        compiler_params=pltpu.CompilerParams(
            dimension_semantics=("arbitrary",)),
    )(x, Wg, bg.reshape(1, -1), W1p, b1.reshape(1, EH), g1.reshape(1, EH),
      be1.reshape(1, EH), W2p, b2)
    return out, aux[0, 0]
